# baseline (speedup 1.0000x reference)
import jax
import jax.numpy as jnp
from jax import lax
from jax.experimental import pallas as pl
from jax.experimental.pallas import tpu as pltpu
from jax.experimental.pallas import tpu_sc as plsc

B, S, D = 4, 2048, 768
N = B * S
EPS = 1e-12

_info = plsc.get_sparse_core_info()
NC, NS = _info.num_cores, _info.num_subcores
NW = NC * NS
TOK_PER_W = N // NW
CHUNK = 16
NCHUNK = TOK_PER_W // CHUNK
NBUF = 8
AHEAD = 4


def _sc_gather(word_hbm, ids_hbm, out_hbm, idx_v, *rest):
    bufs = rest[:NBUF]
    gsems = rest[NBUF:2 * NBUF]
    ssems = rest[2 * NBUF:3 * NBUF]
    wid = lax.axis_index("s") * NC + lax.axis_index("c")
    base = wid * TOK_PER_W
    w_per_row = S // TOK_PER_W
    pltpu.sync_copy(
        ids_hbm.at[wid // w_per_row,
                   pl.ds((wid % w_per_row) * TOK_PER_W, TOK_PER_W)], idx_v)

    def gather(k):
        return pltpu.async_copy(
            word_hbm.at[idx_v.at[pl.ds(k * CHUNK, CHUNK)]], bufs[k % NBUF],
            gsems[k % NBUF])

    gathers = [None] * NCHUNK
    scatters = [None] * NCHUNK
    waited = [False] * NCHUNK
    for k in range(min(AHEAD, NCHUNK)):
        gathers[k] = gather(k)
    for j in range(NCHUNK):
        k = j + AHEAD
        if k < NCHUNK:
            if k - NBUF >= 0:
                scatters[k - NBUF].wait()
                waited[k - NBUF] = True
            gathers[k] = gather(k)
        gathers[j].wait()
        scatters[j] = pltpu.async_copy(
            bufs[j % NBUF], out_hbm.at[pl.ds(base + j * CHUNK, CHUNK)],
            ssems[j % NBUF])
    for j in range(NCHUNK):
        if not waited[j]:
            scatters[j].wait()


def _gather_rows(word_embeddings, ids):
    mesh = plsc.VectorSubcoreMesh(core_axis_name="c", subcore_axis_name="s")
    return pl.kernel(
        _sc_gather,
        mesh=mesh,
        out_type=jax.ShapeDtypeStruct((N, D), jnp.float32),
        scratch_types=[pltpu.VMEM((TOK_PER_W,), jnp.int32)]
        + [pltpu.VMEM((CHUNK, D), jnp.float32) for _ in range(NBUF)]
        + [pltpu.SemaphoreType.DMA for _ in range(2 * NBUF)],
    )(word_embeddings, ids)


def _tc_epilogue(g_ref, p_ref, tt_tab_ref, tt_ref, gamma_ref, beta_ref, o_ref):
    x = g_ref[...] + p_ref[...]
    tt = tt_ref[0].astype(jnp.float32)
    row0 = tt_tab_ref[0:1, :]
    row1 = tt_tab_ref[1:2, :]
    x = x + row0 + tt * (row1 - row0)
    mean = jnp.mean(x, axis=-1, keepdims=True)
    d = x - mean
    var = jnp.mean(d * d, axis=-1, keepdims=True)
    o_ref[...] = (d * lax.rsqrt(var + EPS) * gamma_ref[...][None, :]
                  + beta_ref[...][None, :])


def _epilogue(gathered, position_embeddings, token_type_embeddings, tt_ids,
              ln_gamma, ln_beta):
    tt3 = tt_ids.reshape(B, S, 1).astype(jnp.int8)
    return pl.pallas_call(
        _tc_epilogue,
        grid=(B,),
        in_specs=[
            pl.BlockSpec((S, D), lambda b: (b, 0)),
            pl.BlockSpec((S, D), lambda b: (0, 0)),
            pl.BlockSpec((2, D), lambda b: (0, 0)),
            pl.BlockSpec((1, S, 1), lambda b: (b, 0, 0)),
            pl.BlockSpec((D,), lambda b: (0,)),
            pl.BlockSpec((D,), lambda b: (0,)),
        ],
        out_specs=pl.BlockSpec((S, D), lambda b: (b, 0)),
        out_shape=jax.ShapeDtypeStruct((N, D), jnp.float32),
        input_output_aliases={0: 0},
    )(gathered, position_embeddings, token_type_embeddings, tt3,
      ln_gamma, ln_beta)


def kernel(input_ids, token_type_ids, word_embeddings, position_embeddings,
           token_type_embeddings, ln_gamma, ln_beta):
    ids = input_ids.astype(jnp.int32)
    tt_ids = token_type_ids.astype(jnp.int32)
    g = _gather_rows(word_embeddings, ids)
    out = _epilogue(g, position_embeddings, token_type_embeddings, tt_ids,
                    ln_gamma, ln_beta)
    return out.reshape(B, S, D)
